# baseline (speedup 1.0000x reference)
import functools

import jax
import jax.numpy as jnp
from jax import lax
from jax.experimental import pallas as pl
from jax.experimental.pallas import tpu as pltpu
from jax.experimental.pallas import tpu_sc as plsc

N = 16384
C = 10
NC = 2
NS = 16
L = 16
NW = NC * NS
BPW = N // NW
OUT_W = BPW * C


def _onehot_body(label_hbm, out_hbm, lbl_v, out_v):
    c = lax.axis_index("c")
    s = lax.axis_index("s")
    wid = s * NC + c
    base = wid * BPW

    pltpu.sync_copy(label_hbm.at[pl.ds(base, BPW)], lbl_v)

    zeros = jnp.zeros((L,), jnp.float32)

    def zero_body(k, carry):
        out_v[pl.ds(k * L, L)] = zeros
        return carry

    lax.fori_loop(0, OUT_W // L, zero_body, 0)

    ones = jnp.ones((L,), jnp.float32)
    lane_row = lax.iota(jnp.int32, L) * C
    for j in range(BPW // L):
        lbl = lbl_v[pl.ds(j * L, L)]
        flat = lbl + (lane_row + j * L * C)
        plsc.store_scatter(out_v, [flat], ones)

    pltpu.sync_copy(out_v, out_hbm.at[pl.ds(base * C, OUT_W)])


_onehot_sc = functools.partial(
    pl.kernel,
    mesh=plsc.VectorSubcoreMesh(core_axis_name="c", subcore_axis_name="s"),
    out_type=jax.ShapeDtypeStruct((N * C,), jnp.float32),
    scratch_types=[
        pltpu.VMEM((BPW,), jnp.int32),
        pltpu.VMEM((OUT_W,), jnp.float32),
    ],
    compiler_params=pltpu.CompilerParams(needs_layout_passes=False),
)(_onehot_body)


@jax.jit
def kernel(label):
    return _onehot_sc(label).reshape(N, C)


# speedup vs baseline: 1.1689x; 1.1689x over previous
import functools

import jax
import jax.numpy as jnp
from jax import lax
from jax.experimental import pallas as pl
from jax.experimental.pallas import tpu as pltpu
from jax.experimental.pallas import tpu_sc as plsc

N = 16384
C = 10
NC = 2
NS = 16
L = 16
NW = NC * NS
BPW = N // NW


def _onehot_body(label_hbm, out_hbm, lbl_v, stage_v):
    c = lax.axis_index("c")
    s = lax.axis_index("s")
    wid = s * NC + c
    base = wid * BPW

    pltpu.sync_copy(label_hbm.at[pl.ds(base, BPW)], lbl_v)

    zeros = jnp.zeros((L,), jnp.float32)
    ones = jnp.ones((L,), jnp.float32)
    lane = lax.iota(jnp.int32, L)
    for g in range(BPW // L):
        rows = lane + g * L
        for col in range(C):
            plsc.store_scatter(stage_v, [rows, jnp.full((L,), col, jnp.int32)],
                               zeros)
        lbl = lbl_v[pl.ds(g * L, L)]
        plsc.store_scatter(stage_v, [rows, lbl], ones)

    pltpu.sync_copy(stage_v, out_hbm.at[pl.ds(base, BPW)])


_onehot_sc = functools.partial(
    pl.kernel,
    mesh=plsc.VectorSubcoreMesh(core_axis_name="c", subcore_axis_name="s"),
    out_type=jax.ShapeDtypeStruct((N, C), jnp.float32),
    scratch_types=[
        pltpu.VMEM((BPW,), jnp.int32),
        pltpu.VMEM((BPW, C), jnp.float32),
    ],
    compiler_params=pltpu.CompilerParams(
        needs_layout_passes=False,
        use_tc_tiling_on_sc=False,
        skip_device_barrier=True,
    ),
)(_onehot_body)


@jax.jit
def kernel(label):
    return _onehot_sc(label)


# speedup vs baseline: 2.9935x; 2.5609x over previous
import functools

import jax
import jax.numpy as jnp
from jax.experimental import pallas as pl
from jax.experimental.pallas import tpu as pltpu

N = 16384
C = 10
BLK = 2048
GRID = N // BLK


def _onehot_block(lbl_ref, out_ref):
    lbl = lbl_ref[...]
    classes = jax.lax.broadcasted_iota(jnp.int32, (BLK, C), 1)
    out_ref[...] = jnp.where(
        lbl.reshape(BLK, 1) == classes, 1.0, 0.0
    ).astype(jnp.float32)


_onehot_tc = pl.pallas_call(
    _onehot_block,
    grid=(GRID,),
    in_specs=[pl.BlockSpec((BLK,), lambda i: (i,))],
    out_specs=pl.BlockSpec((BLK, C), lambda i: (i, 0)),
    out_shape=jax.ShapeDtypeStruct((N, C), jnp.float32),
    compiler_params=pltpu.CompilerParams(
        dimension_semantics=("arbitrary",),
    ),
)


@jax.jit
def kernel(label):
    return _onehot_tc(label)


# speedup vs baseline: 13.2201x; 4.4163x over previous
import jax
import jax.numpy as jnp
from jax.experimental import pallas as pl
from jax.experimental.pallas import tpu as pltpu

N = 16384
C = 10
GRID = 4
BLK = N // GRID


def _onehot_block(lbl_ref, out_ref):
    lbl = lbl_ref[...]
    classes = jax.lax.broadcasted_iota(jnp.int32, (C, BLK), 0)
    out_ref[...] = jnp.where(lbl[None, :] == classes, 1.0, 0.0).astype(
        jnp.float32
    )


_onehot_tc = pl.pallas_call(
    _onehot_block,
    grid=(GRID,),
    in_specs=[pl.BlockSpec((BLK,), lambda i: (i,))],
    out_specs=pl.BlockSpec((C, BLK), lambda i: (0, i)),
    out_shape=jax.ShapeDtypeStruct((C, N), jnp.float32),
    compiler_params=pltpu.CompilerParams(
        dimension_semantics=("arbitrary",),
    ),
)


@jax.jit
def kernel(label):
    return _onehot_tc(label).T
